# Initial kernel scaffold; baseline (speedup 1.0000x reference)
import jax
import jax.numpy as jnp
from jax.experimental import pallas as pl


def kernel(numerical_fields, categorical_fields, tables, wide_W1, wide_b1, wide_W2, wide_b2, deep_W1, deep_b1, deep_W2, deep_b2, deep_W3, deep_b3, deep_W4, deep_b4, bias):
    raise NotImplementedError("write your pallas kernel here")


# speedup vs baseline: 1.9692x; 1.9692x over previous
import jax
import jax.numpy as jnp
from jax.experimental import pallas as pl
from jax.experimental.pallas import tpu as pltpu
from jax.experimental.pallas import tpu_sc as plsc

_GW = 512
_BT = 512


def _sc_gather(tables_flat, idx_flat):
    n = idx_flat.shape[1]
    e = tables_flat.shape[1]
    mesh = plsc.VectorSubcoreMesh(core_axis_name="core", subcore_axis_name="subcore")

    @pl.kernel(
        out_type=jax.ShapeDtypeStruct((n, e), jnp.float32),
        mesh=mesh,
        compiler_params=pltpu.CompilerParams(use_tc_tiling_on_sc=False),
    )
    def k(tab_hbm, i_hbm, o_hbm):
        def body(i_vmem, o_vmem):
            pltpu.sync_copy(tab_hbm.at[i_vmem.at[0]], o_vmem)

        pltpu.emit_pipeline(
            body,
            grid=(n // _GW,),
            in_specs=[pl.BlockSpec((1, _GW), lambda i: (0, i))],
            out_specs=[pl.BlockSpec((_GW, e), lambda i: (i, 0))],
            core_axis_name=("core", "subcore"),
            dimension_semantics=(pltpu.PARALLEL,),
        )(i_hbm, o_hbm)

    return k(tables_flat, idx_flat)


def _mlp_body(num_ref, gath_ref, wW1n_ref, wW1e_ref, wb1_ref, wW2_ref,
              dW1n_ref, dW1e_ref, db1_ref, dW2_ref, db2_ref, dW3_ref,
              db3_ref, dW4_ref, cbias_ref, out_ref):
    fn = num_ref[...]
    fe = gath_ref[...]
    h = jnp.dot(fn, wW1n_ref[...], preferred_element_type=jnp.float32)
    h = h + jnp.dot(fe, wW1e_ref[...], preferred_element_type=jnp.float32)
    h = jnp.maximum(h + wb1_ref[...], 0.0)
    wide = jnp.dot(h, wW2_ref[...], preferred_element_type=jnp.float32)
    d = jnp.dot(fn, dW1n_ref[...], preferred_element_type=jnp.float32)
    d = d + jnp.dot(fe, dW1e_ref[...], preferred_element_type=jnp.float32)
    d = jnp.maximum(d + db1_ref[...], 0.0)
    d = jnp.maximum(
        jnp.dot(d, dW2_ref[...], preferred_element_type=jnp.float32) + db2_ref[...], 0.0)
    d = jnp.maximum(
        jnp.dot(d, dW3_ref[...], preferred_element_type=jnp.float32) + db3_ref[...], 0.0)
    deep = jnp.dot(d, dW4_ref[...], preferred_element_type=jnp.float32)
    out_ref[...] = wide + deep + cbias_ref[...]


def _mlp(num, gath, wW1n, wW1e, wb1, wW2, dW1n, dW1e, db1, dW2, db2, dW3,
         db3, dW4, cbias):
    b = num.shape[0]
    grid = (b // _BT,)
    full = lambda shape: pl.BlockSpec(shape, lambda i: (0, 0))
    return pl.pallas_call(
        _mlp_body,
        grid=grid,
        in_specs=[
            pl.BlockSpec((_BT, num.shape[1]), lambda i: (i, 0)),
            pl.BlockSpec((_BT, gath.shape[1]), lambda i: (i, 0)),
            full(wW1n.shape), full(wW1e.shape), full(wb1.shape),
            full(wW2.shape), full(dW1n.shape), full(dW1e.shape),
            full(db1.shape), full(dW2.shape), full(db2.shape),
            full(dW3.shape), full(db3.shape), full(dW4.shape),
            full(cbias.shape),
        ],
        out_specs=pl.BlockSpec((_BT, 1), lambda i: (i, 0)),
        out_shape=jax.ShapeDtypeStruct((b, 1), jnp.float32),
    )(num, gath, wW1n, wW1e, wb1, wW2, dW1n, dW1e, db1, dW2, db2, dW3,
      db3, dW4, cbias)


def kernel(numerical_fields, categorical_fields, tables,
           wide_W1, wide_b1, wide_W2, wide_b2,
           deep_W1, deep_b1, deep_W2, deep_b2,
           deep_W3, deep_b3, deep_W4, deep_b4, bias):
    b, num = numerical_fields.shape
    cat, v, e = tables.shape
    tables_flat = tables.reshape(cat * v, e)
    offs = (jnp.arange(cat, dtype=jnp.int32) * v)[None, :]
    idx_flat = (categorical_fields + offs).reshape(1, b * cat)
    gath = _sc_gather(tables_flat, idx_flat).reshape(b, cat * e)

    cbias = (wide_b2 + deep_b4 + bias).reshape(1, 1)
    out = _mlp(
        numerical_fields, gath,
        wide_W1[:num], wide_W1[num:], wide_b1.reshape(1, -1), wide_W2,
        deep_W1[:num], deep_W1[num:], deep_b1.reshape(1, -1), deep_W2,
        deep_b2.reshape(1, -1), deep_W3, deep_b3.reshape(1, -1), deep_W4,
        cbias)
    return out
